# baseline (speedup 1.0000x reference)
import functools

import jax
import jax.numpy as jnp
from jax.experimental import pallas as pl
from jax.experimental.pallas import tpu as pltpu
from jax.experimental.pallas import tpu_sc as plsc

_B, _N, _M, _C = 4, 8192, 2048, 64
_TN = 2048
_PB = 1

_NSC = 32
_QPW = (_PB * _N) // _NSC
_QCH = 256
_NCH = _QPW // _QCH
_RPC = 3 * _QCH


def _knn_body(u_ref, k_ref, idx_ref, w_ref):
    u = u_ref[0]
    kp = k_ref[0]

    d2 = jnp.zeros((_TN, _M), jnp.float32)
    for d in range(3):
        diff = u[:, d][:, None] - kp[d, :][None, :]
        d2 = d2 + diff * diff

    iota = jax.lax.broadcasted_iota(jnp.int32, (_TN, _M), 1)
    dcur = d2
    vals, idxs = [], []
    for k in range(3):
        mn = jnp.min(dcur, axis=1, keepdims=True)
        am = jnp.min(jnp.where(dcur == mn, iota, _M), axis=1, keepdims=True)
        vals.append(mn)
        idxs.append(am)
        if k < 2:
            dcur = jnp.where(iota == am, jnp.float32(jnp.inf), dcur)

    recips = [1.0 / (jnp.sqrt(jnp.maximum(v, 0.0)) + 1e-8) for v in vals]
    norm = (recips[0] + recips[1]) + recips[2]

    b = pl.program_id(0)
    idx_rows = [jnp.reshape(i + b * _M, (1, _TN)) for i in idxs]
    w_rows = [jnp.reshape(r / norm, (1, _TN)) for r in recips]
    zi = jnp.zeros((5, _TN), jnp.int32)
    zw = jnp.zeros((5, _TN), jnp.float32)
    idx_ref[0] = jnp.concatenate(idx_rows + [zi], axis=0)
    w_ref[0] = jnp.concatenate(w_rows + [zw], axis=0)


def _interp_body(idx_hbm, w_hbm, f_hbm, out_hbm, idx_v, w_v, rows_v,
                 out_scr, sem):
    wid = jax.lax.axis_index("s") * 2 + jax.lax.axis_index("c")
    pb = wid // (_NSC // _PB)
    qbase = (wid % (_NSC // _PB)) * _QPW

    lanes = jax.lax.iota(jnp.int32, 16)
    rowv = [cb * 16 + lanes for cb in range(_C // 16)]

    for ci in range(_NCH):
        q0 = qbase + ci * _QCH
        for k in range(3):
            pltpu.sync_copy(idx_hbm.at[pb, k, pl.ds(q0, _QCH)],
                            idx_v.at[pl.ds(k * _QCH, _QCH)])
            pltpu.sync_copy(w_hbm.at[pb, k, pl.ds(q0, _QCH)],
                            w_v.at[pl.ds(k * _QCH, _QCH)])
        cps = [pltpu.async_copy(f_hbm.at[idx_v.at[pl.ds(j * 128, 128)]],
                                rows_v.at[pl.ds(j * 128, 128)], sem)
               for j in range(_RPC // 128)]
        for cp in cps:
            cp.wait()

        def group(g, carry):
            for j in range(16):
                q = g * 16 + j
                w0 = plsc.load_gather(
                    w_v, [jnp.full((16,), q, jnp.int32)])
                w1 = plsc.load_gather(
                    w_v, [jnp.full((16,), _QCH + q, jnp.int32)])
                w2 = plsc.load_gather(
                    w_v, [jnp.full((16,), 2 * _QCH + q, jnp.int32)])
                qv = jnp.full((16,), q, jnp.int32)
                for cb in range(_C // 16):
                    sl = pl.ds(cb * 16, 16)
                    acc = (w0 * rows_v[q, sl]
                           + w1 * rows_v[_QCH + q, sl]) \
                        + w2 * rows_v[2 * _QCH + q, sl]
                    plsc.store_scatter(out_scr, [rowv[cb], qv], acc)
            return carry

        jax.lax.fori_loop(0, _QCH // 16, group, 0)
        pltpu.sync_copy(out_scr.at[:, pl.ds(0, _QCH)],
                        out_hbm.at[pb, :, pl.ds(q0, _QCH)])


def kernel(unknown, known, known_feats):
    known_t = jnp.transpose(known, (0, 2, 1))
    feats_t = jnp.transpose(known_feats, (0, 2, 1))

    knn = pl.pallas_call(
        _knn_body,
        grid=(_PB, _N // _TN),
        in_specs=[
            pl.BlockSpec((1, _TN, 3), lambda b, i: (b, i, 0)),
            pl.BlockSpec((1, 3, _M), lambda b, i: (b, 0, 0)),
        ],
        out_specs=[
            pl.BlockSpec((1, 8, _TN), lambda b, i: (b, 0, i)),
            pl.BlockSpec((1, 8, _TN), lambda b, i: (b, 0, i)),
        ],
        out_shape=[
            jax.ShapeDtypeStruct((_PB, 8, _N), jnp.int32),
            jax.ShapeDtypeStruct((_PB, 8, _N), jnp.float32),
        ],
    )

    mesh = plsc.VectorSubcoreMesh(core_axis_name="c", subcore_axis_name="s")
    interp = functools.partial(
        pl.kernel,
        mesh=mesh,
        compiler_params=pltpu.CompilerParams(needs_layout_passes=False,
                                             use_tc_tiling_on_sc=False),
        out_type=jax.ShapeDtypeStruct((_PB, _C, _N), jnp.float32),
        scratch_types=[
            pltpu.VMEM((_RPC,), jnp.int32),
            pltpu.VMEM((_RPC,), jnp.float32),
            pltpu.VMEM((_RPC, _C), jnp.float32),
            pltpu.VMEM((_C, _QCH + 1), jnp.float32),
            pltpu.SemaphoreType.DMA,
        ],
    )(_interp_body)

    outs = []
    for p in range(_B // _PB):
        sl = slice(p * _PB, (p + 1) * _PB)
        idx, wgt = knn(unknown[sl], known_t[sl])
        outs.append(interp(idx, wgt, feats_t[sl].reshape(_PB * _M, _C)))
    return jnp.concatenate(outs, axis=0)


# speedup vs baseline: 1.0652x; 1.0652x over previous
import functools

import jax
import jax.numpy as jnp
from jax.experimental import pallas as pl
from jax.experimental.pallas import tpu as pltpu
from jax.experimental.pallas import tpu_sc as plsc

_B, _N, _M, _C = 4, 8192, 2048, 64
_TN = 2048
_PB = 2

_NSC = 32
_QPW = (_PB * _N) // _NSC
_QCH = 256
_NCH = _QPW // _QCH
_RPC = 3 * _QCH


def _knn_body(u_ref, k_ref, idx_ref, w_ref):
    u = u_ref[0]
    kp = k_ref[0]

    d2 = jnp.zeros((_TN, _M), jnp.float32)
    for d in range(3):
        diff = u[:, d][:, None] - kp[d, :][None, :]
        d2 = d2 + diff * diff

    iota = jax.lax.broadcasted_iota(jnp.int32, (_TN, _M), 1)
    dcur = d2
    vals, idxs = [], []
    for k in range(3):
        mn = jnp.min(dcur, axis=1, keepdims=True)
        am = jnp.min(jnp.where(dcur == mn, iota, _M), axis=1, keepdims=True)
        vals.append(mn)
        idxs.append(am)
        if k < 2:
            dcur = jnp.where(iota == am, jnp.float32(jnp.inf), dcur)

    recips = [1.0 / (jnp.sqrt(jnp.maximum(v, 0.0)) + 1e-8) for v in vals]
    norm = (recips[0] + recips[1]) + recips[2]

    b = pl.program_id(0)
    idx_rows = [jnp.reshape(i + b * _M, (1, _TN)) for i in idxs]
    w_rows = [jnp.reshape(r / norm, (1, _TN)) for r in recips]
    zi = jnp.zeros((5, _TN), jnp.int32)
    zw = jnp.zeros((5, _TN), jnp.float32)
    idx_ref[0] = jnp.concatenate(idx_rows + [zi], axis=0)
    w_ref[0] = jnp.concatenate(w_rows + [zw], axis=0)


def _interp_body(idx_hbm, w_hbm, f_hbm, out_hbm, idx_v, w_v, rows_v,
                 out_scr, sem):
    wid = jax.lax.axis_index("s") * 2 + jax.lax.axis_index("c")
    pb = wid // (_NSC // _PB)
    qbase = (wid % (_NSC // _PB)) * _QPW

    lanes = jax.lax.iota(jnp.int32, 16)
    rowv = [cb * 16 + lanes for cb in range(_C // 16)]

    for ci in range(_NCH):
        q0 = qbase + ci * _QCH
        for k in range(3):
            pltpu.sync_copy(idx_hbm.at[pb, k, pl.ds(q0, _QCH)],
                            idx_v.at[pl.ds(k * _QCH, _QCH)])
            pltpu.sync_copy(w_hbm.at[pb, k, pl.ds(q0, _QCH)],
                            w_v.at[pl.ds(k * _QCH, _QCH)])
        cps = [pltpu.async_copy(f_hbm.at[idx_v.at[pl.ds(j * 128, 128)]],
                                rows_v.at[pl.ds(j * 128, 128)], sem)
               for j in range(_RPC // 128)]
        for cp in cps:
            cp.wait()

        def group(g, carry):
            for j in range(16):
                q = g * 16 + j
                w0 = plsc.load_gather(
                    w_v, [jnp.full((16,), q, jnp.int32)])
                w1 = plsc.load_gather(
                    w_v, [jnp.full((16,), _QCH + q, jnp.int32)])
                w2 = plsc.load_gather(
                    w_v, [jnp.full((16,), 2 * _QCH + q, jnp.int32)])
                qv = jnp.full((16,), q, jnp.int32)
                for cb in range(_C // 16):
                    sl = pl.ds(cb * 16, 16)
                    acc = (w0 * rows_v[q, sl]
                           + w1 * rows_v[_QCH + q, sl]) \
                        + w2 * rows_v[2 * _QCH + q, sl]
                    plsc.store_scatter(out_scr, [rowv[cb], qv], acc)
            return carry

        jax.lax.fori_loop(0, _QCH // 16, group, 0)
        pltpu.sync_copy(out_scr.at[:, pl.ds(0, _QCH)],
                        out_hbm.at[pb, :, pl.ds(q0, _QCH)])


def kernel(unknown, known, known_feats):
    known_t = jnp.transpose(known, (0, 2, 1))
    feats_t = jnp.transpose(known_feats, (0, 2, 1))

    knn = pl.pallas_call(
        _knn_body,
        grid=(_PB, _N // _TN),
        in_specs=[
            pl.BlockSpec((1, _TN, 3), lambda b, i: (b, i, 0)),
            pl.BlockSpec((1, 3, _M), lambda b, i: (b, 0, 0)),
        ],
        out_specs=[
            pl.BlockSpec((1, 8, _TN), lambda b, i: (b, 0, i)),
            pl.BlockSpec((1, 8, _TN), lambda b, i: (b, 0, i)),
        ],
        out_shape=[
            jax.ShapeDtypeStruct((_PB, 8, _N), jnp.int32),
            jax.ShapeDtypeStruct((_PB, 8, _N), jnp.float32),
        ],
    )

    mesh = plsc.VectorSubcoreMesh(core_axis_name="c", subcore_axis_name="s")
    interp = functools.partial(
        pl.kernel,
        mesh=mesh,
        compiler_params=pltpu.CompilerParams(needs_layout_passes=False,
                                             use_tc_tiling_on_sc=False),
        out_type=jax.ShapeDtypeStruct((_PB, _C, _N), jnp.float32),
        scratch_types=[
            pltpu.VMEM((_RPC,), jnp.int32),
            pltpu.VMEM((_RPC,), jnp.float32),
            pltpu.VMEM((_RPC, _C), jnp.float32),
            pltpu.VMEM((_C, _QCH + 1), jnp.float32),
            pltpu.SemaphoreType.DMA,
        ],
    )(_interp_body)

    outs = []
    for p in range(_B // _PB):
        sl = slice(p * _PB, (p + 1) * _PB)
        idx, wgt = knn(unknown[sl], known_t[sl])
        outs.append(interp(idx, wgt, feats_t[sl].reshape(_PB * _M, _C)))
    return jnp.concatenate(outs, axis=0)
